# baseline (speedup 1.0000x reference)
import functools

import jax
import jax.numpy as jnp
from jax.experimental import pallas as pl
from jax.experimental.pallas import tpu as pltpu

BLOCK_N = 2048
NBUF = 4


def _matmul_bias_kernel(xt_hbm, w_ref, b_ref, o_ref, buf, sems):
    n = o_ref.shape[1]
    nblk = n // BLOCK_N

    def copy_in(blk, slot):
        return pltpu.make_async_copy(
            xt_hbm.at[:, pl.ds(blk * BLOCK_N, BLOCK_N)],
            buf.at[slot],
            sems.at[slot],
        )

    for j in range(min(NBUF, nblk)):
        copy_in(j, j).start()

    for i in range(nblk):
        slot = i % NBUF
        copy_in(i, slot).wait()
        o_ref[:, pl.ds(i * BLOCK_N, BLOCK_N)] = (
            jnp.dot(w_ref[...], buf[slot], preferred_element_type=jnp.float32)
            + b_ref[...]
        )
        if i + NBUF < nblk:
            copy_in(i + NBUF, slot).start()


@jax.jit
def kernel(x, W_last, b_last, W_dom, b_dom):
    xs = jnp.squeeze(x)
    n, k = xs.shape
    m = W_last.shape[0]
    xt = jnp.swapaxes(xs, 0, 1)
    bc = b_last.reshape(m, 1)
    out_t = pl.pallas_call(
        _matmul_bias_kernel,
        in_specs=[
            pl.BlockSpec(memory_space=pltpu.MemorySpace.HBM),
            pl.BlockSpec((m, k), lambda: (0, 0)),
            pl.BlockSpec((m, 1), lambda: (0, 0)),
        ],
        out_specs=pl.BlockSpec((m, n), lambda: (0, 0)),
        out_shape=jax.ShapeDtypeStruct((m, n), jnp.float32),
        scratch_shapes=[
            pltpu.VMEM((NBUF, 864, BLOCK_N), jnp.float32),
            pltpu.SemaphoreType.DMA((NBUF,)),
        ],
    )(xt, W_last, bc)
    return jnp.swapaxes(out_t, 0, 1)
